# baseline (speedup 1.0000x reference)
import functools

import jax
import jax.numpy as jnp
import numpy as np
from jax import lax
from jax.experimental import pallas as pl
from jax.experimental.pallas import tpu as pltpu
from jax.experimental.pallas import tpu_sc as plsc

N = 10000
E = 320000
NODE_IN = 55
D = 128
DW = D // 2
L = 4

NC = 2
NS = 16

CH = 80
EDGES_PER_CORE = E // NC
EDGES_PER_TILE = EDGES_PER_CORE // NS
CHUNKS = EDGES_PER_TILE // CH
N_PAD = 10240
ROWS_PER_TILE = N_PAD // NS

BN = 2000
BE = 10000

_PERM = np.concatenate([
    32 * g + np.concatenate([np.arange(0, 32, 2), np.arange(1, 32, 2)])
    for g in range(D // 32)
])

_HI = np.int32(-65536)


def _pack_bf16_pairs(ye, yo):
    be = lax.bitcast_convert_type(ye, jnp.int32) + 0x8000
    bo = lax.bitcast_convert_type(yo, jnp.int32) + 0x8000
    return lax.shift_right_logical(be, 16) | (bo & _HI)


def _proj_body(nf_ref, win_ref, bin_ref, wn_ref, h_ref, hw_ref):
    h = jnp.maximum(
        jnp.dot(nf_ref[...], win_ref[...], preferred_element_type=jnp.float32)
        + bin_ref[...], 0.0)
    h_ref[...] = h
    hw_ref[...] = jnp.dot(h, wn_ref[...], preferred_element_type=jnp.float32)


def _project(node_feats, W_in, b_in, Wn_perm):
    return pl.pallas_call(
        _proj_body,
        grid=(N // BN,),
        in_specs=[
            pl.BlockSpec((BN, NODE_IN), lambda i: (i, 0)),
            pl.BlockSpec((NODE_IN, D), lambda i: (0, 0)),
            pl.BlockSpec((1, D), lambda i: (0, 0)),
            pl.BlockSpec((D, D), lambda i: (0, 0)),
        ],
        out_specs=[
            pl.BlockSpec((BN, D), lambda i: (i, 0)),
            pl.BlockSpec((BN, D), lambda i: (i, 0)),
        ],
        out_shape=[
            jax.ShapeDtypeStruct((N, D), jnp.float32),
            jax.ShapeDtypeStruct((N, D), jnp.float32),
        ],
    )(node_feats, W_in, b_in.reshape(1, D), Wn_perm)


def _edge_body(efa_ref, efb_ref, w2_ref, out_ref):
    ya = jnp.dot(efa_ref[...], w2_ref[0], preferred_element_type=jnp.float32)
    yb = jnp.dot(efb_ref[...], w2_ref[0], preferred_element_type=jnp.float32)
    a = jnp.concatenate([ya[:, :DW], yb[:, :DW]], axis=1)
    b = jnp.concatenate([ya[:, DW:], yb[:, DW:]], axis=1)
    out_ref[...] = _pack_bf16_pairs(a, b)


def _edge_transform(ef_a, ef_b, W2l):
    ein = ef_a.shape[1]
    be2 = BE // 2
    return pl.pallas_call(
        _edge_body,
        grid=((E // 2) // be2,),
        in_specs=[
            pl.BlockSpec((be2, ein), lambda e: (e, 0)),
            pl.BlockSpec((be2, ein), lambda e: (e, 0)),
            pl.BlockSpec((1, ein, D), lambda e: (0, 0, 0)),
        ],
        out_specs=pl.BlockSpec((be2, D), lambda e: (e, 0)),
        out_shape=jax.ShapeDtypeStruct((E // 2, D), jnp.int32),
    )(ef_a, ef_b, W2l)


def _upd_body(h_ref, p_ref, wt_ref, wb_ref, b_ref, wn_ref, hnew_ref, hw_ref):
    p = p_ref[0] + p_ref[1]
    hn = jnp.maximum(
        jnp.dot(h_ref[...], wt_ref[...], preferred_element_type=jnp.float32)
        + jnp.dot(p, wb_ref[...], preferred_element_type=jnp.float32)
        + b_ref[...], 0.0)
    hnew_ref[...] = hn
    hw_ref[...] = jnp.dot(hn, wn_ref[...], preferred_element_type=jnp.float32)


def _update(h, parts, W_top, W_bot_perm, b, Wn_perm):
    return pl.pallas_call(
        _upd_body,
        grid=(N // BN,),
        in_specs=[
            pl.BlockSpec((BN, D), lambda i: (i, 0)),
            pl.BlockSpec((NC, BN, D), lambda i: (0, i, 0)),
            pl.BlockSpec((D, D), lambda i: (0, 0)),
            pl.BlockSpec((D, D), lambda i: (0, 0)),
            pl.BlockSpec((1, D), lambda i: (0, 0)),
            pl.BlockSpec((D, D), lambda i: (0, 0)),
        ],
        out_specs=[
            pl.BlockSpec((BN, D), lambda i: (i, 0)),
            pl.BlockSpec((BN, D), lambda i: (i, 0)),
        ],
        out_shape=[
            jax.ShapeDtypeStruct((N, D), jnp.float32),
            jax.ShapeDtypeStruct((N, D), jnp.float32),
        ],
    )(h, parts, W_top, W_bot_perm, b.reshape(1, D), Wn_perm)


def _sc_body(hw_hbm, ew_hbm, src_hbm, dst_hbm, out_hbm, acc,
             src0, dst0, rows0, ew0, src1, dst1, rows1, ew1,
             gsem0, esem0, ssem0, gsem1, esem1, ssem1):
    c = lax.axis_index("c")
    s = lax.axis_index("s")
    bufs = ((src0, dst0, rows0, ew0, gsem0, esem0, ssem0),
            (src1, dst1, rows1, ew1, gsem1, esem1, ssem1))

    zeros16 = jnp.zeros((16,), jnp.float32)

    def zfill(i, _):
        for j in range(D // 16):
            rows0[i, pl.ds(j * 16, 16)] = zeros16
        return 0

    lax.fori_loop(0, CH, zfill, 0)
    for j in range(ROWS_PER_TILE // CH):
        pltpu.sync_copy(rows0, acc.at[pl.ds(s * ROWS_PER_TILE + j * CH, CH)])
    plsc.subcore_barrier()

    hbase0 = c * (EDGES_PER_CORE // 2) + s * (EDGES_PER_TILE // 2)

    def wait_scatter(b):
        _, dst_v, rows_v, _, _, _, ssem = bufs[b]
        pltpu.make_async_copy(rows_v, acc.at[dst_v], ssem).wait()

    def start(i, b):
        src_v, dst_v, rows_v, ew_v, gsem, esem, _ = bufs[b]
        hbase = hbase0 + i * (CH // 2)
        h2 = CH // 2
        pltpu.sync_copy(src_hbm.at[pl.ds(hbase, h2)], src_v.at[pl.ds(0, h2)])
        pltpu.sync_copy(src_hbm.at[pl.ds(E // 2 + hbase, h2)],
                        src_v.at[pl.ds(h2, h2)])
        pltpu.sync_copy(dst_hbm.at[pl.ds(hbase, h2)], dst_v.at[pl.ds(0, h2)])
        pltpu.sync_copy(dst_hbm.at[pl.ds(E // 2 + hbase, h2)],
                        dst_v.at[pl.ds(h2, h2)])
        pltpu.async_copy(hw_hbm.at[src_v], rows_v, gsem)
        pltpu.async_copy(ew_hbm.at[pl.ds(hbase, CH // 2)], ew_v, esem)

    def finish(i, b):
        src_v, dst_v, rows_v, ew_v, gsem, esem, ssem = bufs[b]
        hbase = hbase0 + i * (CH // 2)
        pltpu.make_async_copy(hw_hbm.at[src_v], rows_v, gsem).wait()
        pltpu.make_async_copy(
            ew_hbm.at[pl.ds(hbase, CH // 2)], ew_v, esem).wait()

        @plsc.parallel_loop(0, CH // 2, 1, unroll=2)
        def mul(pp):
            for de in range(2):
                e = pp + de * (CH // 2)
                for j in range(D // 32):
                    we = ew_v[pp, pl.ds(64 * de + 16 * j, 16)]
                    e_even = lax.bitcast_convert_type(
                        lax.shift_left(we, 16), jnp.float32)
                    e_odd = lax.bitcast_convert_type(we & _HI, jnp.float32)
                    sl_e = pl.ds(32 * j, 16)
                    sl_o = pl.ds(32 * j + 16, 16)
                    rows_v[e, sl_e] = rows_v[e, sl_e] * e_even
                    rows_v[e, sl_o] = rows_v[e, sl_o] * e_odd

        pltpu.async_copy(rows_v, acc.at[dst_v], ssem, add=True)

    start(0, 0)
    start(1, 1)

    def pair(g, _):
        i0 = 2 * g
        finish(i0, 0)

        @pl.when(i0 + 2 < CHUNKS)
        def _():
            wait_scatter(0)
            start(i0 + 2, 0)

        @pl.when(i0 + 1 < CHUNKS)
        def _():
            finish(i0 + 1, 1)

        @pl.when(i0 + 3 < CHUNKS)
        def _():
            wait_scatter(1)
            start(i0 + 3, 1)

        return 0

    lax.fori_loop(0, (CHUNKS + 1) // 2, pair, 0)
    wait_scatter(0)
    wait_scatter(1)
    plsc.subcore_barrier()

    pltpu.sync_copy(acc.at[pl.ds(s * ROWS_PER_TILE, ROWS_PER_TILE)],
                    out_hbm.at[c, pl.ds(s * ROWS_PER_TILE, ROWS_PER_TILE)])


def _sc_message_pass(hw, ew_pairs, src, dst):
    mesh = plsc.VectorSubcoreMesh(core_axis_name="c", subcore_axis_name="s")
    return pl.kernel(
        _sc_body,
        out_type=jax.ShapeDtypeStruct((NC, N_PAD, D), jnp.float32),
        mesh=mesh,
        scratch_types=[
            pltpu.VMEM_SHARED((N_PAD, D), jnp.float32),
            pltpu.VMEM((CH,), jnp.int32),
            pltpu.VMEM((CH,), jnp.int32),
            pltpu.VMEM((CH, D), jnp.float32),
            pltpu.VMEM((CH // 2, D), jnp.int32),
            pltpu.VMEM((CH,), jnp.int32),
            pltpu.VMEM((CH,), jnp.int32),
            pltpu.VMEM((CH, D), jnp.float32),
            pltpu.VMEM((CH // 2, D), jnp.int32),
            pltpu.SemaphoreType.DMA,
            pltpu.SemaphoreType.DMA,
            pltpu.SemaphoreType.DMA,
            pltpu.SemaphoreType.DMA,
            pltpu.SemaphoreType.DMA,
            pltpu.SemaphoreType.DMA,
        ],
    )(hw, ew_pairs, src, dst)


def kernel(node_feats, edge_feats, edge_index, W_in, b_in, W_node, W_edge,
           W_new, b_new):
    src = edge_index[0]
    dst = edge_index[1]
    perm = _PERM
    ef_a, ef_b = edge_feats[:E // 2], edge_feats[E // 2:]
    W2 = jnp.concatenate([W_edge[:, :, 0::2], W_edge[:, :, 1::2]], axis=2)
    ew = [_edge_transform(ef_a, ef_b, W2[l:l + 1]) for l in range(L)]
    h, hw = _project(node_feats, W_in, b_in, W_node[0][:, perm])
    for l in range(L):
        parts = _sc_message_pass(hw, ew[l], src, dst)
        wn_next = W_node[(l + 1) % L][:, perm]
        h, hw = _update(h, parts, W_new[l][:D], W_new[l][D:][perm],
                        b_new[l], wn_next)
    return h


# speedup vs baseline: 1.2786x; 1.2786x over previous
import functools

import jax
import jax.numpy as jnp
import numpy as np
from jax import lax
from jax.experimental import pallas as pl
from jax.experimental.pallas import tpu as pltpu
from jax.experimental.pallas import tpu_sc as plsc

N = 10000
E = 320000
NODE_IN = 55
D = 128
DW = D // 2
L = 4

NC = 2
NS = 16

CH = 80
EDGES_PER_CORE = E // NC
EDGES_PER_TILE = EDGES_PER_CORE // NS
CHUNKS = EDGES_PER_TILE // CH
N_PAD = 10240
ROWS_PER_TILE = N_PAD // NS

BN = 2000
BE = 10000

_PERM = np.concatenate([
    32 * g + np.concatenate([np.arange(0, 32, 2), np.arange(1, 32, 2)])
    for g in range(D // 32)
])

_HI = np.int32(-65536)


def _pack_bf16_pairs(ye, yo):
    be = lax.bitcast_convert_type(ye, jnp.int32) + 0x8000
    bo = lax.bitcast_convert_type(yo, jnp.int32) + 0x8000
    return lax.shift_right_logical(be, 16) | (bo & _HI)


def _proj_body(nf_ref, win_ref, bin_ref, wn_ref, h_ref, hw_ref):
    h = jnp.maximum(
        jnp.dot(nf_ref[...], win_ref[...], preferred_element_type=jnp.float32)
        + bin_ref[...], 0.0)
    h_ref[...] = h
    hw_ref[...] = jnp.dot(h, wn_ref[...], preferred_element_type=jnp.float32)


def _project(node_feats, W_in, b_in, Wn_perm):
    return pl.pallas_call(
        _proj_body,
        grid=(N // BN,),
        in_specs=[
            pl.BlockSpec((BN, NODE_IN), lambda i: (i, 0)),
            pl.BlockSpec((NODE_IN, D), lambda i: (0, 0)),
            pl.BlockSpec((1, D), lambda i: (0, 0)),
            pl.BlockSpec((D, D), lambda i: (0, 0)),
        ],
        out_specs=[
            pl.BlockSpec((BN, D), lambda i: (i, 0)),
            pl.BlockSpec((BN, D), lambda i: (i, 0)),
        ],
        out_shape=[
            jax.ShapeDtypeStruct((N, D), jnp.float32),
            jax.ShapeDtypeStruct((N, D), jnp.float32),
        ],
    )(node_feats, W_in, b_in.reshape(1, D), Wn_perm)


def _edge_body(efa_ref, efb_ref, w2_ref, out_ref):
    ya = jnp.dot(efa_ref[...], w2_ref[0], preferred_element_type=jnp.float32)
    yb = jnp.dot(efb_ref[...], w2_ref[0], preferred_element_type=jnp.float32)
    a = jnp.concatenate([ya[:, :DW], yb[:, :DW]], axis=1)
    b = jnp.concatenate([ya[:, DW:], yb[:, DW:]], axis=1)
    out_ref[...] = _pack_bf16_pairs(a, b)


def _edge_transform(ef_a, ef_b, W2l):
    ein = ef_a.shape[1]
    be2 = BE // 2
    return pl.pallas_call(
        _edge_body,
        grid=((E // 2) // be2,),
        in_specs=[
            pl.BlockSpec((be2, ein), lambda e: (e, 0)),
            pl.BlockSpec((be2, ein), lambda e: (e, 0)),
            pl.BlockSpec((1, ein, D), lambda e: (0, 0, 0)),
        ],
        out_specs=pl.BlockSpec((be2, D), lambda e: (e, 0)),
        out_shape=jax.ShapeDtypeStruct((E // 2, D), jnp.int32),
    )(ef_a, ef_b, W2l)


def _upd_body(h_ref, p_ref, wt_ref, wb_ref, b_ref, wn_ref, hnew_ref, hw_ref):
    p = p_ref[0] + p_ref[1]
    hn = jnp.maximum(
        jnp.dot(h_ref[...], wt_ref[...], preferred_element_type=jnp.float32)
        + jnp.dot(p, wb_ref[...], preferred_element_type=jnp.float32)
        + b_ref[...], 0.0)
    hnew_ref[...] = hn
    hw_ref[...] = jnp.dot(hn, wn_ref[...], preferred_element_type=jnp.float32)


def _update(h, parts, W_top, W_bot_perm, b, Wn_perm):
    return pl.pallas_call(
        _upd_body,
        grid=(N // BN,),
        in_specs=[
            pl.BlockSpec((BN, D), lambda i: (i, 0)),
            pl.BlockSpec((NC, BN, D), lambda i: (0, i, 0)),
            pl.BlockSpec((D, D), lambda i: (0, 0)),
            pl.BlockSpec((D, D), lambda i: (0, 0)),
            pl.BlockSpec((1, D), lambda i: (0, 0)),
            pl.BlockSpec((D, D), lambda i: (0, 0)),
        ],
        out_specs=[
            pl.BlockSpec((BN, D), lambda i: (i, 0)),
            pl.BlockSpec((BN, D), lambda i: (i, 0)),
        ],
        out_shape=[
            jax.ShapeDtypeStruct((N, D), jnp.float32),
            jax.ShapeDtypeStruct((N, D), jnp.float32),
        ],
    )(h, parts, W_top, W_bot_perm, b.reshape(1, D), Wn_perm)


def _sc_body(hw_hbm, ew_hbm, src_hbm, dst_hbm, out_hbm, acc,
             dst_all, src0, rows0, ew0, src1, rows1, ew1,
             gsem0, esem0, ssem0, gsem1, esem1, ssem1):
    c = lax.axis_index("c")
    s = lax.axis_index("s")
    w = c * NS + s
    bufs = ((src0, rows0, ew0, gsem0, esem0, ssem0),
            (src1, rows1, ew1, gsem1, esem1, ssem1))

    pltpu.sync_copy(dst_hbm.at[w], dst_all)

    zeros16 = jnp.zeros((16,), jnp.float32)

    def zfill(i, _):
        for j in range(D // 16):
            rows0[i, pl.ds(j * 16, 16)] = zeros16
        return 0

    lax.fori_loop(0, CH, zfill, 0)
    for j in range(ROWS_PER_TILE // CH):
        pltpu.sync_copy(rows0, acc.at[pl.ds(s * ROWS_PER_TILE + j * CH, CH)])
    plsc.subcore_barrier()

    hbase0 = (c * (EDGES_PER_CORE // 2) + s * (EDGES_PER_TILE // 2))

    def wait_scatter(i, b):
        _, rows_v, _, _, _, ssem = bufs[b]
        pltpu.make_async_copy(rows_v, acc.at[dst_all.at[i, 0]], ssem).wait()

    def start(i, b):
        src_v, rows_v, ew_v, gsem, esem, _ = bufs[b]
        hbase = hbase0 + i * (CH // 2)
        h2 = CH // 2
        pltpu.sync_copy(src_hbm.at[pl.ds(hbase, h2)], src_v.at[pl.ds(0, h2)])
        pltpu.sync_copy(src_hbm.at[pl.ds(E // 2 + hbase, h2)],
                        src_v.at[pl.ds(h2, h2)])
        pltpu.async_copy(hw_hbm.at[src_v], rows_v, gsem)
        pltpu.async_copy(ew_hbm.at[pl.ds(hbase, CH // 2)], ew_v, esem)

    def finish(i, b):
        src_v, rows_v, ew_v, gsem, esem, ssem = bufs[b]
        hbase = hbase0 + i * (CH // 2)
        pltpu.make_async_copy(hw_hbm.at[src_v], rows_v, gsem).wait()
        pltpu.make_async_copy(
            ew_hbm.at[pl.ds(hbase, CH // 2)], ew_v, esem).wait()

        @plsc.parallel_loop(0, CH // 2, 1, unroll=2)
        def mul(pp):
            for de in range(2):
                e = pp + de * (CH // 2)
                for j in range(D // 32):
                    we = ew_v[pp, pl.ds(64 * de + 16 * j, 16)]
                    e_even = lax.bitcast_convert_type(
                        lax.shift_left(we, 16), jnp.float32)
                    e_odd = lax.bitcast_convert_type(we & _HI, jnp.float32)
                    sl_e = pl.ds(32 * j, 16)
                    sl_o = pl.ds(32 * j + 16, 16)
                    rows_v[e, sl_e] = rows_v[e, sl_e] * e_even
                    rows_v[e, sl_o] = rows_v[e, sl_o] * e_odd

        pltpu.async_copy(rows_v, acc.at[dst_all.at[i, 0]], ssem, add=True)

    start(0, 0)
    start(1, 1)

    def pair(g, _):
        i0 = 2 * g
        finish(i0, 0)

        @pl.when(i0 + 2 < CHUNKS)
        def _():
            wait_scatter(i0, 0)
            start(i0 + 2, 0)

        @pl.when(i0 + 1 < CHUNKS)
        def _():
            finish(i0 + 1, 1)

        @pl.when(i0 + 3 < CHUNKS)
        def _():
            wait_scatter(i0 + 1, 1)
            start(i0 + 3, 1)

        return 0

    lax.fori_loop(0, (CHUNKS + 1) // 2, pair, 0)
    wait_scatter(CHUNKS - 1, 0)
    wait_scatter(CHUNKS - 2, 1)
    plsc.subcore_barrier()

    pltpu.sync_copy(acc.at[pl.ds(s * ROWS_PER_TILE, ROWS_PER_TILE)],
                    out_hbm.at[c, pl.ds(s * ROWS_PER_TILE, ROWS_PER_TILE)])


def _sc_message_pass(hw, ew_pairs, src1d, dst3):
    mesh = plsc.VectorSubcoreMesh(core_axis_name="c", subcore_axis_name="s")
    return pl.kernel(
        _sc_body,
        out_type=jax.ShapeDtypeStruct((NC, N_PAD, D), jnp.float32),
        mesh=mesh,
        scratch_types=[
            pltpu.VMEM_SHARED((N_PAD, D), jnp.float32),
            pltpu.VMEM((CHUNKS, 1, CH), jnp.int32),
            pltpu.VMEM((CH,), jnp.int32),
            pltpu.VMEM((CH, D), jnp.float32),
            pltpu.VMEM((CH // 2, D), jnp.int32),
            pltpu.VMEM((CH,), jnp.int32),
            pltpu.VMEM((CH, D), jnp.float32),
            pltpu.VMEM((CH // 2, D), jnp.int32),
            pltpu.SemaphoreType.DMA,
            pltpu.SemaphoreType.DMA,
            pltpu.SemaphoreType.DMA,
            pltpu.SemaphoreType.DMA,
            pltpu.SemaphoreType.DMA,
            pltpu.SemaphoreType.DMA,
        ],
    )(hw, ew_pairs, src1d, dst3)


def kernel(node_feats, edge_feats, edge_index, W_in, b_in, W_node, W_edge,
           W_new, b_new):
    src = edge_index[0]
    da = edge_index[1, :E // 2].reshape(NC * NS, CHUNKS, 1, CH // 2)
    db = edge_index[1, E // 2:].reshape(NC * NS, CHUNKS, 1, CH // 2)
    dst3 = jnp.concatenate([da, db], axis=3)
    perm = _PERM
    ef_a, ef_b = edge_feats[:E // 2], edge_feats[E // 2:]
    W2 = jnp.concatenate([W_edge[:, :, 0::2], W_edge[:, :, 1::2]], axis=2)
    ew = [_edge_transform(ef_a, ef_b, W2[l:l + 1]) for l in range(L)]
    h, hw = _project(node_feats, W_in, b_in, W_node[0][:, perm])
    for l in range(L):
        parts = _sc_message_pass(hw, ew[l], src, dst3)
        wn_next = W_node[(l + 1) % L][:, perm]
        h, hw = _update(h, parts, W_new[l][:D], W_new[l][D:][perm],
                        b_new[l], wn_next)
    return h


# speedup vs baseline: 1.4668x; 1.1472x over previous
import functools

import jax
import jax.numpy as jnp
import numpy as np
from jax import lax
from jax.experimental import pallas as pl
from jax.experimental.pallas import tpu as pltpu
from jax.experimental.pallas import tpu_sc as plsc

N = 10000
E = 320000
NODE_IN = 55
D = 128
DW = D // 2
L = 4

NC = 2
NS = 16

CH = 80
EDGES_PER_CORE = E // NC
EDGES_PER_TILE = EDGES_PER_CORE // NS
CHUNKS = EDGES_PER_TILE // CH
N_PAD = 10240
ROWS_PER_TILE = N_PAD // NS

BN = 2000
BE = 10000

_PERM = np.concatenate([
    32 * g + np.concatenate([np.arange(0, 32, 2), np.arange(1, 32, 2)])
    for g in range(D // 32)
])

_HI = np.int32(-65536)


def _pack_bf16_pairs(ye, yo):
    be = lax.bitcast_convert_type(ye, jnp.int32) + 0x8000
    bo = lax.bitcast_convert_type(yo, jnp.int32) + 0x8000
    return lax.shift_right_logical(be, 16) | (bo & _HI)


def _proj_body(nf_ref, win_ref, bin_ref, wn_ref, h_ref, hw_ref):
    h = jnp.maximum(
        jnp.dot(nf_ref[...], win_ref[...], preferred_element_type=jnp.float32)
        + bin_ref[...], 0.0)
    h_ref[...] = h
    hw_ref[...] = jnp.dot(h, wn_ref[...], preferred_element_type=jnp.float32)


def _project(node_feats, W_in, b_in, Wn_perm):
    return pl.pallas_call(
        _proj_body,
        grid=(N // BN,),
        in_specs=[
            pl.BlockSpec((BN, NODE_IN), lambda i: (i, 0)),
            pl.BlockSpec((NODE_IN, D), lambda i: (0, 0)),
            pl.BlockSpec((1, D), lambda i: (0, 0)),
            pl.BlockSpec((D, D), lambda i: (0, 0)),
        ],
        out_specs=[
            pl.BlockSpec((BN, D), lambda i: (i, 0)),
            pl.BlockSpec((BN, D), lambda i: (i, 0)),
        ],
        out_shape=[
            jax.ShapeDtypeStruct((N, D), jnp.float32),
            jax.ShapeDtypeStruct((N, D), jnp.float32),
        ],
    )(node_feats, W_in, b_in.reshape(1, D), Wn_perm)


def _edge_body(efa_ref, efb_ref, w2_ref, out_ref):
    ya = jnp.dot(efa_ref[...], w2_ref[0], preferred_element_type=jnp.float32)
    yb = jnp.dot(efb_ref[...], w2_ref[0], preferred_element_type=jnp.float32)
    a = jnp.concatenate([ya[:, :DW], yb[:, :DW]], axis=1)
    b = jnp.concatenate([ya[:, DW:], yb[:, DW:]], axis=1)
    out_ref[...] = _pack_bf16_pairs(a, b)


def _edge_transform(ef_a, ef_b, W2l):
    ein = ef_a.shape[1]
    be2 = BE // 2
    return pl.pallas_call(
        _edge_body,
        grid=((E // 2) // be2,),
        in_specs=[
            pl.BlockSpec((be2, ein), lambda e: (e, 0)),
            pl.BlockSpec((be2, ein), lambda e: (e, 0)),
            pl.BlockSpec((1, ein, D), lambda e: (0, 0, 0)),
        ],
        out_specs=pl.BlockSpec((be2, D), lambda e: (e, 0)),
        out_shape=jax.ShapeDtypeStruct((E // 2, D), jnp.int32),
    )(ef_a, ef_b, W2l)


def _upd_body(h_ref, p_ref, wt_ref, wb_ref, b_ref, wn_ref, hnew_ref, hw_ref):
    p = p_ref[0] + p_ref[1]
    hn = jnp.maximum(
        jnp.dot(h_ref[...], wt_ref[...], preferred_element_type=jnp.float32)
        + jnp.dot(p, wb_ref[...], preferred_element_type=jnp.float32)
        + b_ref[...], 0.0)
    hnew_ref[...] = hn
    hw_ref[...] = jnp.dot(hn, wn_ref[...], preferred_element_type=jnp.float32)


def _update(h, parts, W_top, W_bot_perm, b, Wn_perm):
    return pl.pallas_call(
        _upd_body,
        grid=(N // BN,),
        in_specs=[
            pl.BlockSpec((BN, D), lambda i: (i, 0)),
            pl.BlockSpec((NC, BN, D), lambda i: (0, i, 0)),
            pl.BlockSpec((D, D), lambda i: (0, 0)),
            pl.BlockSpec((D, D), lambda i: (0, 0)),
            pl.BlockSpec((1, D), lambda i: (0, 0)),
            pl.BlockSpec((D, D), lambda i: (0, 0)),
        ],
        out_specs=[
            pl.BlockSpec((BN, D), lambda i: (i, 0)),
            pl.BlockSpec((BN, D), lambda i: (i, 0)),
        ],
        out_shape=[
            jax.ShapeDtypeStruct((N, D), jnp.float32),
            jax.ShapeDtypeStruct((N, D), jnp.float32),
        ],
    )(h, parts, W_top, W_bot_perm, b.reshape(1, D), Wn_perm)


def _sc_body(hw_hbm, ew_hbm, src_hbm, dst_hbm, out_hbm, acc,
             dst_all, src0, rows0, ew0, src1, rows1, ew1,
             gsem0, esem0, ssem0, gsem1, esem1, ssem1):
    c = lax.axis_index("c")
    s = lax.axis_index("s")
    w = c * NS + s
    bufs = ((src0, rows0, ew0, gsem0, esem0, ssem0),
            (src1, rows1, ew1, gsem1, esem1, ssem1))

    pltpu.sync_copy(dst_hbm.at[w], dst_all)

    zeros16 = jnp.zeros((16,), jnp.float32)

    def zfill(i, _):
        for j in range(D // 16):
            rows0[i, pl.ds(j * 16, 16)] = zeros16
        return 0

    lax.fori_loop(0, CH, zfill, 0)
    for j in range(ROWS_PER_TILE // CH):
        pltpu.sync_copy(rows0, acc.at[pl.ds(s * ROWS_PER_TILE + j * CH, CH)])
    plsc.subcore_barrier()

    hbase0 = (c * (EDGES_PER_CORE // 2) + s * (EDGES_PER_TILE // 2))

    def wait_scatter(i, b):
        _, rows_v, _, _, _, ssem = bufs[b]
        pltpu.make_async_copy(rows_v, acc.at[dst_all.at[i, 0]], ssem).wait()

    def start(i, b):
        src_v, rows_v, ew_v, gsem, esem, _ = bufs[b]
        hbase = hbase0 + i * (CH // 2)
        pltpu.sync_copy(src_hbm.at[w, i], src_v)
        pltpu.async_copy(hw_hbm.at[src_v.at[0]], rows_v, gsem)
        pltpu.async_copy(ew_hbm.at[pl.ds(hbase, CH // 2)], ew_v, esem)

    def finish(i, b):
        src_v, rows_v, ew_v, gsem, esem, ssem = bufs[b]
        hbase = hbase0 + i * (CH // 2)
        pltpu.make_async_copy(hw_hbm.at[src_v.at[0]], rows_v, gsem).wait()
        pltpu.make_async_copy(
            ew_hbm.at[pl.ds(hbase, CH // 2)], ew_v, esem).wait()

        @plsc.parallel_loop(0, CH // 2, 1, unroll=2)
        def mul(pp):
            for de in range(2):
                e = pp + de * (CH // 2)
                for j in range(D // 32):
                    we = ew_v[pp, pl.ds(64 * de + 16 * j, 16)]
                    e_even = lax.bitcast_convert_type(
                        lax.shift_left(we, 16), jnp.float32)
                    e_odd = lax.bitcast_convert_type(we & _HI, jnp.float32)
                    sl_e = pl.ds(32 * j, 16)
                    sl_o = pl.ds(32 * j + 16, 16)
                    rows_v[e, sl_e] = rows_v[e, sl_e] * e_even
                    rows_v[e, sl_o] = rows_v[e, sl_o] * e_odd

        pltpu.async_copy(rows_v, acc.at[dst_all.at[i, 0]], ssem, add=True)

    start(0, 0)
    start(1, 1)

    def pair(g, _):
        i0 = 2 * g
        finish(i0, 0)

        @pl.when(i0 + 2 < CHUNKS)
        def _():
            wait_scatter(i0, 0)
            start(i0 + 2, 0)

        @pl.when(i0 + 1 < CHUNKS)
        def _():
            finish(i0 + 1, 1)

        @pl.when(i0 + 3 < CHUNKS)
        def _():
            wait_scatter(i0 + 1, 1)
            start(i0 + 3, 1)

        return 0

    lax.fori_loop(0, (CHUNKS + 1) // 2, pair, 0)
    wait_scatter(CHUNKS - 1, 0)
    wait_scatter(CHUNKS - 2, 1)
    plsc.subcore_barrier()

    pltpu.sync_copy(acc.at[pl.ds(s * ROWS_PER_TILE, ROWS_PER_TILE)],
                    out_hbm.at[c, pl.ds(s * ROWS_PER_TILE, ROWS_PER_TILE)])


def _sc_message_pass(hw, ew_pairs, src1d, dst3):
    mesh = plsc.VectorSubcoreMesh(core_axis_name="c", subcore_axis_name="s")
    return pl.kernel(
        _sc_body,
        out_type=jax.ShapeDtypeStruct((NC, N_PAD, D), jnp.float32),
        mesh=mesh,
        scratch_types=[
            pltpu.VMEM_SHARED((N_PAD, D), jnp.float32),
            pltpu.VMEM((CHUNKS, 1, CH), jnp.int32),
            pltpu.VMEM((1, CH), jnp.int32),
            pltpu.VMEM((CH, D), jnp.float32),
            pltpu.VMEM((CH // 2, D), jnp.int32),
            pltpu.VMEM((1, CH), jnp.int32),
            pltpu.VMEM((CH, D), jnp.float32),
            pltpu.VMEM((CH // 2, D), jnp.int32),
            pltpu.SemaphoreType.DMA,
            pltpu.SemaphoreType.DMA,
            pltpu.SemaphoreType.DMA,
            pltpu.SemaphoreType.DMA,
            pltpu.SemaphoreType.DMA,
            pltpu.SemaphoreType.DMA,
        ],
    )(hw, ew_pairs, src1d, dst3)


def kernel(node_feats, edge_feats, edge_index, W_in, b_in, W_node, W_edge,
           W_new, b_new):
    sa = edge_index[0, :E // 2].reshape(NC * NS, CHUNKS, 1, CH // 2)
    sb = edge_index[0, E // 2:].reshape(NC * NS, CHUNKS, 1, CH // 2)
    src = jnp.concatenate([sa, sb], axis=3)
    da = edge_index[1, :E // 2].reshape(NC * NS, CHUNKS, 1, CH // 2)
    db = edge_index[1, E // 2:].reshape(NC * NS, CHUNKS, 1, CH // 2)
    dst3 = jnp.concatenate([da, db], axis=3)
    perm = _PERM
    ef_a, ef_b = edge_feats[:E // 2], edge_feats[E // 2:]
    W2 = jnp.concatenate([W_edge[:, :, 0::2], W_edge[:, :, 1::2]], axis=2)
    ew = [_edge_transform(ef_a, ef_b, W2[l:l + 1]) for l in range(L)]
    h, hw = _project(node_feats, W_in, b_in, W_node[0][:, perm])
    for l in range(L):
        parts = _sc_message_pass(hw, ew[l], src, dst3)
        wn_next = W_node[(l + 1) % L][:, perm]
        h, hw = _update(h, parts, W_new[l][:D], W_new[l][D:][perm],
                        b_new[l], wn_next)
    return h


# speedup vs baseline: 1.4741x; 1.0050x over previous
import functools

import jax
import jax.numpy as jnp
import numpy as np
from jax import lax
from jax.experimental import pallas as pl
from jax.experimental.pallas import tpu as pltpu
from jax.experimental.pallas import tpu_sc as plsc

N = 10000
E = 320000
NODE_IN = 55
D = 128
DW = D // 2
L = 4

NC = 2
NS = 16

CH = 80
EDGES_PER_CORE = E // NC
EDGES_PER_TILE = EDGES_PER_CORE // NS
CHUNKS = EDGES_PER_TILE // CH
N_PAD = 10240
ROWS_PER_TILE = N_PAD // NS

BN = 2000
BE = 10000

_PERM = np.concatenate([
    32 * g + np.concatenate([np.arange(0, 32, 2), np.arange(1, 32, 2)])
    for g in range(D // 32)
])

_HI = np.int32(-65536)


def _pack_bf16_pairs(ye, yo):
    be = lax.bitcast_convert_type(ye, jnp.int32) + 0x8000
    bo = lax.bitcast_convert_type(yo, jnp.int32) + 0x8000
    return lax.shift_right_logical(be, 16) | (bo & _HI)


def _proj_body(nf_ref, win_ref, bin_ref, wn_ref, h_ref, hw_ref):
    h = jnp.maximum(
        jnp.dot(nf_ref[...], win_ref[...], preferred_element_type=jnp.float32)
        + bin_ref[...], 0.0)
    h_ref[...] = h
    hw_ref[...] = jnp.dot(h, wn_ref[...], preferred_element_type=jnp.float32)


def _project(node_feats, W_in, b_in, Wn_perm):
    return pl.pallas_call(
        _proj_body,
        grid=(N // BN,),
        in_specs=[
            pl.BlockSpec((BN, NODE_IN), lambda i: (i, 0)),
            pl.BlockSpec((NODE_IN, D), lambda i: (0, 0)),
            pl.BlockSpec((1, D), lambda i: (0, 0)),
            pl.BlockSpec((D, D), lambda i: (0, 0)),
        ],
        out_specs=[
            pl.BlockSpec((BN, D), lambda i: (i, 0)),
            pl.BlockSpec((BN, D), lambda i: (i, 0)),
        ],
        out_shape=[
            jax.ShapeDtypeStruct((N, D), jnp.float32),
            jax.ShapeDtypeStruct((N, D), jnp.float32),
        ],
    )(node_feats, W_in, b_in.reshape(1, D), Wn_perm)


def _edge_body(efa_ref, efb_ref, w2_ref, out_ref):
    ya = jnp.dot(efa_ref[...], w2_ref[0], preferred_element_type=jnp.float32)
    yb = jnp.dot(efb_ref[...], w2_ref[0], preferred_element_type=jnp.float32)
    a = jnp.concatenate([ya[:, :DW], yb[:, :DW]], axis=1)
    b = jnp.concatenate([ya[:, DW:], yb[:, DW:]], axis=1)
    out_ref[...] = _pack_bf16_pairs(a, b)


def _edge_transform(ef_a, ef_b, W2l):
    ein = ef_a.shape[1]
    be2 = BE // 2
    return pl.pallas_call(
        _edge_body,
        grid=((E // 2) // be2,),
        in_specs=[
            pl.BlockSpec((be2, ein), lambda e: (e, 0)),
            pl.BlockSpec((be2, ein), lambda e: (e, 0)),
            pl.BlockSpec((1, ein, D), lambda e: (0, 0, 0)),
        ],
        out_specs=pl.BlockSpec((be2, D), lambda e: (e, 0)),
        out_shape=jax.ShapeDtypeStruct((E // 2, D), jnp.int32),
    )(ef_a, ef_b, W2l)


def _upd_body(h_ref, p_ref, wt_ref, wb_ref, b_ref, wn_ref, hnew_ref, hw_ref):
    p = p_ref[0] + p_ref[1]
    hn = jnp.maximum(
        jnp.dot(h_ref[...], wt_ref[...], preferred_element_type=jnp.float32)
        + jnp.dot(p, wb_ref[...], preferred_element_type=jnp.float32)
        + b_ref[...], 0.0)
    hnew_ref[...] = hn
    hw_ref[...] = jnp.dot(hn, wn_ref[...], preferred_element_type=jnp.float32)


def _update(h, parts, W_top, W_bot_perm, b, Wn_perm):
    return pl.pallas_call(
        _upd_body,
        grid=(N // BN,),
        in_specs=[
            pl.BlockSpec((BN, D), lambda i: (i, 0)),
            pl.BlockSpec((NC, BN, D), lambda i: (0, i, 0)),
            pl.BlockSpec((D, D), lambda i: (0, 0)),
            pl.BlockSpec((D, D), lambda i: (0, 0)),
            pl.BlockSpec((1, D), lambda i: (0, 0)),
            pl.BlockSpec((D, D), lambda i: (0, 0)),
        ],
        out_specs=[
            pl.BlockSpec((BN, D), lambda i: (i, 0)),
            pl.BlockSpec((BN, D), lambda i: (i, 0)),
        ],
        out_shape=[
            jax.ShapeDtypeStruct((N, D), jnp.float32),
            jax.ShapeDtypeStruct((N, D), jnp.float32),
        ],
    )(h, parts, W_top, W_bot_perm, b.reshape(1, D), Wn_perm)


def _sc_body(hw_hbm, ew_hbm, src_hbm, dst_hbm, out_hbm, acc,
             dst_all, src0, rows0, ew0, src1, rows1, ew1,
             gsem0, esem0, ssem0, gsem1, esem1, ssem1):
    c = lax.axis_index("c")
    s = lax.axis_index("s")
    w = c * NS + s
    bufs = ((src0, rows0, ew0, gsem0, esem0, ssem0),
            (src1, rows1, ew1, gsem1, esem1, ssem1))

    pltpu.sync_copy(dst_hbm.at[w], dst_all)

    zeros16 = jnp.zeros((16,), jnp.float32)

    def zfill(i, _):
        for j in range(D // 16):
            rows0[i, pl.ds(j * 16, 16)] = zeros16
        return 0

    lax.fori_loop(0, CH, zfill, 0)
    for j in range(ROWS_PER_TILE // CH):
        pltpu.sync_copy(rows0, acc.at[pl.ds(s * ROWS_PER_TILE + j * CH, CH)])
    plsc.subcore_barrier()

    hbase0 = (c * (EDGES_PER_CORE // 2) + s * (EDGES_PER_TILE // 2))

    def wait_scatter(i, b):
        _, rows_v, _, _, _, ssem = bufs[b]
        pltpu.make_async_copy(rows_v, acc.at[dst_all.at[i, 0]], ssem).wait()

    def start(i, b):
        src_v, rows_v, ew_v, gsem, esem, _ = bufs[b]
        hbase = hbase0 + i * (CH // 2)
        pltpu.sync_copy(src_hbm.at[w, i], src_v)
        pltpu.async_copy(hw_hbm.at[src_v.at[0]], rows_v, gsem)
        pltpu.async_copy(ew_hbm.at[pl.ds(hbase, CH // 2)], ew_v, esem)

    def finish(i, b):
        src_v, rows_v, ew_v, gsem, esem, ssem = bufs[b]
        hbase = hbase0 + i * (CH // 2)
        pltpu.make_async_copy(hw_hbm.at[src_v.at[0]], rows_v, gsem).wait()
        pltpu.make_async_copy(
            ew_hbm.at[pl.ds(hbase, CH // 2)], ew_v, esem).wait()

        @plsc.parallel_loop(0, CH // 2, 1, unroll=4)
        def mul(pp):
            for de in range(2):
                e = pp + de * (CH // 2)
                for j in range(D // 32):
                    we = ew_v[pp, pl.ds(64 * de + 16 * j, 16)]
                    e_even = lax.bitcast_convert_type(
                        lax.shift_left(we, 16), jnp.float32)
                    e_odd = lax.bitcast_convert_type(we & _HI, jnp.float32)
                    sl_e = pl.ds(32 * j, 16)
                    sl_o = pl.ds(32 * j + 16, 16)
                    rows_v[e, sl_e] = rows_v[e, sl_e] * e_even
                    rows_v[e, sl_o] = rows_v[e, sl_o] * e_odd

        pltpu.async_copy(rows_v, acc.at[dst_all.at[i, 0]], ssem, add=True)

    start(0, 0)
    start(1, 1)

    def pair(g, _):
        i0 = 2 * g
        finish(i0, 0)

        @pl.when(i0 + 2 < CHUNKS)
        def _():
            wait_scatter(i0, 0)
            start(i0 + 2, 0)

        @pl.when(i0 + 1 < CHUNKS)
        def _():
            finish(i0 + 1, 1)

        @pl.when(i0 + 3 < CHUNKS)
        def _():
            wait_scatter(i0 + 1, 1)
            start(i0 + 3, 1)

        return 0

    lax.fori_loop(0, (CHUNKS + 1) // 2, pair, 0)
    wait_scatter(CHUNKS - 1, 0)
    wait_scatter(CHUNKS - 2, 1)
    plsc.subcore_barrier()

    pltpu.sync_copy(acc.at[pl.ds(s * ROWS_PER_TILE, ROWS_PER_TILE)],
                    out_hbm.at[c, pl.ds(s * ROWS_PER_TILE, ROWS_PER_TILE)])


def _sc_message_pass(hw, ew_pairs, src1d, dst3):
    mesh = plsc.VectorSubcoreMesh(core_axis_name="c", subcore_axis_name="s")
    return pl.kernel(
        _sc_body,
        out_type=jax.ShapeDtypeStruct((NC, N_PAD, D), jnp.float32),
        mesh=mesh,
        scratch_types=[
            pltpu.VMEM_SHARED((N_PAD, D), jnp.float32),
            pltpu.VMEM((CHUNKS, 1, CH), jnp.int32),
            pltpu.VMEM((1, CH), jnp.int32),
            pltpu.VMEM((CH, D), jnp.float32),
            pltpu.VMEM((CH // 2, D), jnp.int32),
            pltpu.VMEM((1, CH), jnp.int32),
            pltpu.VMEM((CH, D), jnp.float32),
            pltpu.VMEM((CH // 2, D), jnp.int32),
            pltpu.SemaphoreType.DMA,
            pltpu.SemaphoreType.DMA,
            pltpu.SemaphoreType.DMA,
            pltpu.SemaphoreType.DMA,
            pltpu.SemaphoreType.DMA,
            pltpu.SemaphoreType.DMA,
        ],
    )(hw, ew_pairs, src1d, dst3)


def kernel(node_feats, edge_feats, edge_index, W_in, b_in, W_node, W_edge,
           W_new, b_new):
    sa = edge_index[0, :E // 2].reshape(NC * NS, CHUNKS, 1, CH // 2)
    sb = edge_index[0, E // 2:].reshape(NC * NS, CHUNKS, 1, CH // 2)
    src = jnp.concatenate([sa, sb], axis=3)
    da = edge_index[1, :E // 2].reshape(NC * NS, CHUNKS, 1, CH // 2)
    db = edge_index[1, E // 2:].reshape(NC * NS, CHUNKS, 1, CH // 2)
    dst3 = jnp.concatenate([da, db], axis=3)
    perm = _PERM
    ef_a, ef_b = edge_feats[:E // 2], edge_feats[E // 2:]
    W2 = jnp.concatenate([W_edge[:, :, 0::2], W_edge[:, :, 1::2]], axis=2)
    ew = [_edge_transform(ef_a, ef_b, W2[l:l + 1]) for l in range(L)]
    h, hw = _project(node_feats, W_in, b_in, W_node[0][:, perm])
    for l in range(L):
        parts = _sc_message_pass(hw, ew[l], src, dst3)
        wn_next = W_node[(l + 1) % L][:, perm]
        h, hw = _update(h, parts, W_new[l][:D], W_new[l][D:][perm],
                        b_new[l], wn_next)
    return h


# speedup vs baseline: 1.6516x; 1.1204x over previous
import functools

import jax
import jax.numpy as jnp
import numpy as np
from jax import lax
from jax.experimental import pallas as pl
from jax.experimental.pallas import tpu as pltpu
from jax.experimental.pallas import tpu_sc as plsc

N = 10000
E = 320000
NODE_IN = 55
D = 128
DW = D // 2
L = 4

NC = 2
NS = 16

CH = 80
EDGES_PER_CORE = E // NC
EDGES_PER_TILE = EDGES_PER_CORE // NS
CHUNKS = EDGES_PER_TILE // CH
N_PAD = 10240
ROWS_PER_TILE = N_PAD // NS

BN = 2000
BE = 10000

_PERM = np.concatenate([
    32 * g + np.concatenate([np.arange(0, 32, 2), np.arange(1, 32, 2)])
    for g in range(D // 32)
])

_HI = np.int32(-65536)


def _pack_bf16_pairs(ye, yo):
    be = lax.bitcast_convert_type(ye, jnp.int32) + 0x8000
    bo = lax.bitcast_convert_type(yo, jnp.int32) + 0x8000
    return lax.shift_right_logical(be, 16) | (bo & _HI)


def _proj_body(nf_ref, win_ref, bin_ref, wn_ref, h_ref, hw_ref):
    h = jnp.maximum(
        jnp.dot(nf_ref[...], win_ref[...], preferred_element_type=jnp.float32)
        + bin_ref[...], 0.0)
    h_ref[...] = h
    hw_ref[...] = jnp.dot(h, wn_ref[...], preferred_element_type=jnp.float32)


def _project(node_feats, W_in, b_in, Wn_perm):
    return pl.pallas_call(
        _proj_body,
        grid=(N // BN,),
        in_specs=[
            pl.BlockSpec((BN, NODE_IN), lambda i: (i, 0)),
            pl.BlockSpec((NODE_IN, D), lambda i: (0, 0)),
            pl.BlockSpec((1, D), lambda i: (0, 0)),
            pl.BlockSpec((D, D), lambda i: (0, 0)),
        ],
        out_specs=[
            pl.BlockSpec((BN, D), lambda i: (i, 0)),
            pl.BlockSpec((BN, D), lambda i: (i, 0)),
        ],
        out_shape=[
            jax.ShapeDtypeStruct((N, D), jnp.float32),
            jax.ShapeDtypeStruct((N, D), jnp.float32),
        ],
    )(node_feats, W_in, b_in.reshape(1, D), Wn_perm)


def _edge_body(efa_ref, efb_ref, w2_ref, out_ref):
    ya = jnp.dot(efa_ref[...], w2_ref[0], preferred_element_type=jnp.float32)
    yb = jnp.dot(efb_ref[...], w2_ref[0], preferred_element_type=jnp.float32)
    a = jnp.concatenate([ya[:, :DW], yb[:, :DW]], axis=1)
    b = jnp.concatenate([ya[:, DW:], yb[:, DW:]], axis=1)
    out_ref[...] = _pack_bf16_pairs(a, b)


def _edge_transform(ef_a, ef_b, W2l):
    ein = ef_a.shape[1]
    be2 = BE // 2
    return pl.pallas_call(
        _edge_body,
        grid=((E // 2) // be2,),
        in_specs=[
            pl.BlockSpec((be2, ein), lambda e: (e, 0)),
            pl.BlockSpec((be2, ein), lambda e: (e, 0)),
            pl.BlockSpec((1, ein, D), lambda e: (0, 0, 0)),
        ],
        out_specs=pl.BlockSpec((be2, D), lambda e: (e, 0)),
        out_shape=jax.ShapeDtypeStruct((E // 2, D), jnp.int32),
    )(ef_a, ef_b, W2l)


def _upd_body(h_ref, p_ref, wt_ref, wb_ref, b_ref, wn_ref, hnew_ref, hw_ref):
    p = p_ref[0] + p_ref[1]
    hn = jnp.maximum(
        jnp.dot(h_ref[...], wt_ref[...], preferred_element_type=jnp.float32)
        + jnp.dot(p, wb_ref[...], preferred_element_type=jnp.float32)
        + b_ref[...], 0.0)
    hnew_ref[...] = hn
    hw_ref[...] = jnp.dot(hn, wn_ref[...], preferred_element_type=jnp.float32)


def _update(h, parts, W_top, W_bot_perm, b, Wn_perm):
    return pl.pallas_call(
        _upd_body,
        grid=(N // BN,),
        in_specs=[
            pl.BlockSpec((BN, D), lambda i: (i, 0)),
            pl.BlockSpec((NC, BN, D), lambda i: (0, i, 0)),
            pl.BlockSpec((D, D), lambda i: (0, 0)),
            pl.BlockSpec((D, D), lambda i: (0, 0)),
            pl.BlockSpec((1, D), lambda i: (0, 0)),
            pl.BlockSpec((D, D), lambda i: (0, 0)),
        ],
        out_specs=[
            pl.BlockSpec((BN, D), lambda i: (i, 0)),
            pl.BlockSpec((BN, D), lambda i: (i, 0)),
        ],
        out_shape=[
            jax.ShapeDtypeStruct((N, D), jnp.float32),
            jax.ShapeDtypeStruct((N, D), jnp.float32),
        ],
    )(h, parts, W_top, W_bot_perm, b.reshape(1, D), Wn_perm)


def _sc_body(hw_hbm, ew_hbm, src_hbm, dst_hbm, out_hbm, acc,
             dst_all, src0, rows0, ew0, src1, rows1, ew1,
             gsem0, esem0, ssem0, gsem1, esem1, ssem1, isem0, isem1):
    c = lax.axis_index("c")
    s = lax.axis_index("s")
    w = c * NS + s
    bufs = ((src0, rows0, ew0, gsem0, esem0, ssem0, isem0),
            (src1, rows1, ew1, gsem1, esem1, ssem1, isem1))

    pltpu.sync_copy(dst_hbm.at[w], dst_all)

    zeros16 = jnp.zeros((16,), jnp.float32)

    def zfill(i, _):
        for j in range(D // 16):
            rows0[i, pl.ds(j * 16, 16)] = zeros16
        return 0

    lax.fori_loop(0, CH, zfill, 0)
    for j in range(ROWS_PER_TILE // CH):
        pltpu.sync_copy(rows0, acc.at[pl.ds(s * ROWS_PER_TILE + j * CH, CH)])
    plsc.subcore_barrier()

    hbase0 = (c * (EDGES_PER_CORE // 2) + s * (EDGES_PER_TILE // 2))

    def wait_scatter(i, b):
        _, rows_v, _, _, _, ssem, _ = bufs[b]
        pltpu.make_async_copy(rows_v, acc.at[dst_all.at[i, 0]], ssem).wait()

    def fetch_src(i, b):
        src_v, _, _, _, _, _, isem = bufs[b]
        pltpu.async_copy(src_hbm.at[w, i], src_v, isem)

    def start(i, b):
        src_v, rows_v, ew_v, gsem, esem, _, isem = bufs[b]
        hbase = hbase0 + i * (CH // 2)
        pltpu.make_async_copy(src_hbm.at[w, i], src_v, isem).wait()
        pltpu.async_copy(hw_hbm.at[src_v.at[0]], rows_v, gsem)
        pltpu.async_copy(ew_hbm.at[pl.ds(hbase, CH // 2)], ew_v, esem)

    def finish(i, b):
        src_v, rows_v, ew_v, gsem, esem, ssem, _ = bufs[b]
        hbase = hbase0 + i * (CH // 2)
        pltpu.make_async_copy(hw_hbm.at[src_v.at[0]], rows_v, gsem).wait()

        @pl.when(i + 2 < CHUNKS)
        def _():
            fetch_src(i + 2, b)
        pltpu.make_async_copy(
            ew_hbm.at[pl.ds(hbase, CH // 2)], ew_v, esem).wait()

        @plsc.parallel_loop(0, CH // 2, 1, unroll=4)
        def mul(pp):
            for de in range(2):
                e = pp + de * (CH // 2)
                for j in range(D // 32):
                    we = ew_v[pp, pl.ds(64 * de + 16 * j, 16)]
                    e_even = lax.bitcast_convert_type(
                        lax.shift_left(we, 16), jnp.float32)
                    e_odd = lax.bitcast_convert_type(we & _HI, jnp.float32)
                    sl_e = pl.ds(32 * j, 16)
                    sl_o = pl.ds(32 * j + 16, 16)
                    rows_v[e, sl_e] = rows_v[e, sl_e] * e_even
                    rows_v[e, sl_o] = rows_v[e, sl_o] * e_odd

        pltpu.async_copy(rows_v, acc.at[dst_all.at[i, 0]], ssem, add=True)

    fetch_src(0, 0)
    fetch_src(1, 1)
    start(0, 0)
    start(1, 1)

    def pair(g, _):
        i0 = 2 * g
        finish(i0, 0)

        @pl.when(i0 + 2 < CHUNKS)
        def _():
            wait_scatter(i0, 0)
            start(i0 + 2, 0)

        @pl.when(i0 + 1 < CHUNKS)
        def _():
            finish(i0 + 1, 1)

        @pl.when(i0 + 3 < CHUNKS)
        def _():
            wait_scatter(i0 + 1, 1)
            start(i0 + 3, 1)

        return 0

    lax.fori_loop(0, (CHUNKS + 1) // 2, pair, 0)
    wait_scatter(CHUNKS - 1, 0)
    wait_scatter(CHUNKS - 2, 1)
    plsc.subcore_barrier()

    pltpu.sync_copy(acc.at[pl.ds(s * ROWS_PER_TILE, ROWS_PER_TILE)],
                    out_hbm.at[c, pl.ds(s * ROWS_PER_TILE, ROWS_PER_TILE)])


def _sc_message_pass(hw, ew_pairs, src1d, dst3):
    mesh = plsc.VectorSubcoreMesh(core_axis_name="c", subcore_axis_name="s")
    return pl.kernel(
        _sc_body,
        out_type=jax.ShapeDtypeStruct((NC, N_PAD, D), jnp.float32),
        mesh=mesh,
        scratch_types=[
            pltpu.VMEM_SHARED((N_PAD, D), jnp.float32),
            pltpu.VMEM((CHUNKS, 1, CH), jnp.int32),
            pltpu.VMEM((1, CH), jnp.int32),
            pltpu.VMEM((CH, D), jnp.float32),
            pltpu.VMEM((CH // 2, D), jnp.int32),
            pltpu.VMEM((1, CH), jnp.int32),
            pltpu.VMEM((CH, D), jnp.float32),
            pltpu.VMEM((CH // 2, D), jnp.int32),
            pltpu.SemaphoreType.DMA,
            pltpu.SemaphoreType.DMA,
            pltpu.SemaphoreType.DMA,
            pltpu.SemaphoreType.DMA,
            pltpu.SemaphoreType.DMA,
            pltpu.SemaphoreType.DMA,
            pltpu.SemaphoreType.DMA,
            pltpu.SemaphoreType.DMA,
        ],
    )(hw, ew_pairs, src1d, dst3)


def kernel(node_feats, edge_feats, edge_index, W_in, b_in, W_node, W_edge,
           W_new, b_new):
    sa = edge_index[0, :E // 2].reshape(NC * NS, CHUNKS, 1, CH // 2)
    sb = edge_index[0, E // 2:].reshape(NC * NS, CHUNKS, 1, CH // 2)
    src = jnp.concatenate([sa, sb], axis=3)
    da = edge_index[1, :E // 2].reshape(NC * NS, CHUNKS, 1, CH // 2)
    db = edge_index[1, E // 2:].reshape(NC * NS, CHUNKS, 1, CH // 2)
    dst3 = jnp.concatenate([da, db], axis=3)
    perm = _PERM
    ef_a, ef_b = edge_feats[:E // 2], edge_feats[E // 2:]
    W2 = jnp.concatenate([W_edge[:, :, 0::2], W_edge[:, :, 1::2]], axis=2)
    ew = [_edge_transform(ef_a, ef_b, W2[l:l + 1]) for l in range(L)]
    h, hw = _project(node_feats, W_in, b_in, W_node[0][:, perm])
    for l in range(L):
        parts = _sc_message_pass(hw, ew[l], src, dst3)
        wn_next = W_node[(l + 1) % L][:, perm]
        h, hw = _update(h, parts, W_new[l][:D], W_new[l][D:][perm],
                        b_new[l], wn_next)
    return h
